# Initial kernel scaffold; baseline (speedup 1.0000x reference)
import functools

import jax
import jax.numpy as jnp
from jax import lax
from jax.experimental import pallas as pl
from jax.experimental.pallas import tpu as pltpu
from jax.experimental.pallas import tpu_sc as plsc

N = 10000
E = 320000
D = 128
NPAD = 10240
W_EDGES = 128
N_WIN = E // W_EDGES
N_WORKERS = 32
ROWS_PER_TILE = N // 16
PROWS_PER_TILE = NPAD // 16

_mesh = plsc.VectorSubcoreMesh(core_axis_name="c", subcore_axis_name="s")


def _rsqrt16(d):
    i = plsc.bitcast(d, jnp.int32)
    i = jnp.full((16,), 0x5F3759DF, jnp.int32) - lax.shift_right_logical(i, 1)
    y = plsc.bitcast(i, jnp.float32)
    half = d * 0.5
    for _ in range(3):
        y = y * (1.5 - half * y * y)
    return y


@functools.partial(
    pl.kernel,
    out_type=jax.ShapeDtypeStruct((NPAD,), jnp.float32),
    mesh=_mesh,
    scratch_types=[
        pltpu.VMEM((W_EDGES,), jnp.int32),
        pltpu.VMEM((W_EDGES,), jnp.float32),
        pltpu.VMEM((PROWS_PER_TILE,), jnp.float32),
        pltpu.VMEM((PROWS_PER_TILE,), jnp.float32),
        pltpu.VMEM_SHARED((NPAD,), jnp.float32),
    ],
)
def _deg_dinv(dst_hbm, dinv_hbm, idx_v, ones_v, buf_v, dinv_v, acc_sh):
    c = lax.axis_index("c")
    s = lax.axis_index("s")

    @pl.when(c == 0)
    def _():
        for j in range(W_EDGES // 16):
            ones_v[pl.ds(j * 16, 16)] = jnp.full((16,), 1.0, jnp.float32)

        def zbody(j, _):
            buf_v[pl.ds(j * 16, 16)] = jnp.zeros((16,), jnp.float32)
            return None

        lax.fori_loop(0, PROWS_PER_TILE // 16, zbody, None)
        pltpu.sync_copy(buf_v, acc_sh.at[pl.ds(s * PROWS_PER_TILE, PROWS_PER_TILE)])
        plsc.subcore_barrier()

        lo = (N_WIN * s) // 16
        hi = (N_WIN * (s + 1)) // 16

        def body(i, _):
            pltpu.sync_copy(dst_hbm.at[pl.ds(i * W_EDGES, W_EDGES)], idx_v)
            pltpu.sync_copy(ones_v, acc_sh.at[idx_v], add=True)
            return None

        lax.fori_loop(lo, hi, body, None)
        plsc.subcore_barrier()

        r0 = s * PROWS_PER_TILE
        pltpu.sync_copy(acc_sh.at[pl.ds(r0, PROWS_PER_TILE)], buf_v)

        def dbody(j, _):
            d = buf_v[pl.ds(j * 16, 16)] + 1.0
            dinv_v[pl.ds(j * 16, 16)] = _rsqrt16(d)
            return None

        lax.fori_loop(0, PROWS_PER_TILE // 16, dbody, None)
        pltpu.sync_copy(dinv_v, dinv_hbm.at[pl.ds(r0, PROWS_PER_TILE)])


@functools.partial(
    pl.kernel,
    out_type=jax.ShapeDtypeStruct((2, N, D), jnp.float32),
    mesh=_mesh,
    scratch_types=[
        pltpu.VMEM((W_EDGES,), jnp.int32),
        pltpu.VMEM((W_EDGES,), jnp.int32),
        pltpu.VMEM((W_EDGES, D), jnp.float32),
        pltpu.VMEM_SHARED((N, D), jnp.float32),
        pltpu.SemaphoreType.DMA,
    ],
)
def _aggregate(hp_hbm, src_hbm, dst_hbm, p_hbm, src_v, dst_v, rows_v, acc_sh, sem):
    c = lax.axis_index("c")
    s = lax.axis_index("s")
    r0 = s * ROWS_PER_TILE
    pltpu.sync_copy(hp_hbm.at[pl.ds(r0, ROWS_PER_TILE)],
                    acc_sh.at[pl.ds(r0, ROWS_PER_TILE)])
    plsc.subcore_barrier()

    wid = c * 16 + s
    lo = (N_WIN * wid) // N_WORKERS
    hi = (N_WIN * (wid + 1)) // N_WORKERS

    def body(i, _):
        base = i * W_EDGES
        pltpu.sync_copy(src_hbm.at[pl.ds(base, W_EDGES)], src_v)
        pltpu.sync_copy(dst_hbm.at[pl.ds(base, W_EDGES)], dst_v)
        pltpu.async_copy(hp_hbm.at[src_v], rows_v, sem).wait()
        pltpu.sync_copy(rows_v, acc_sh.at[dst_v], add=True)
        return None

    lax.fori_loop(lo, hi, body, None)
    plsc.subcore_barrier()
    pltpu.sync_copy(acc_sh.at[pl.ds(r0, ROWS_PER_TILE)],
                    p_hbm.at[c, pl.ds(r0, ROWS_PER_TILE)])


def _mm1_body(x_ref, dinv_ref, w_ref, o_ref):
    o_ref[...] = jnp.dot(x_ref[...] * dinv_ref[...], w_ref[...],
                         preferred_element_type=jnp.float32)


def _mid_body(p_ref, hp_ref, dinv_ref, b_ref, w_ref, o_ref):
    agg = p_ref[0] + p_ref[1] - hp_ref[...]
    x2 = jnp.maximum(agg * dinv_ref[...] + b_ref[...], 0.0) * dinv_ref[...]
    o_ref[...] = jnp.dot(x2, w_ref[...], preferred_element_type=jnp.float32)


def _fin_body(p_ref, hp_ref, dinv_ref, b_ref, o_ref):
    o_ref[...] = (p_ref[0] + p_ref[1] - hp_ref[...]) * dinv_ref[...] + b_ref[...]


_out_nd = jax.ShapeDtypeStruct((N, D), jnp.float32)
_mm1 = pl.pallas_call(_mm1_body, out_shape=_out_nd)
_mid = pl.pallas_call(_mid_body, out_shape=_out_nd)
_fin = pl.pallas_call(_fin_body, out_shape=_out_nd)


def kernel(x, edge_index, W1, b1, W2, b2):
    src = edge_index[0].astype(jnp.int32)
    dst = edge_index[1].astype(jnp.int32)
    b1r = b1.reshape(1, D)
    b2r = b2.reshape(1, D)

    dinv = _deg_dinv(dst)[:N].reshape(N, 1)
    h1p = _mm1(x, dinv, W1)
    p1 = _aggregate(h1p, src, dst)
    h2p = _mid(p1, h1p, dinv, b1r, W2)
    p2 = _aggregate(h2p, src, dst)
    return _fin(p2, h2p, dinv, b2r)


# speedup vs baseline: 15.5560x; 15.5560x over previous
import functools

import jax
import jax.numpy as jnp
from jax import lax
from jax.experimental import pallas as pl
from jax.experimental.pallas import tpu as pltpu
from jax.experimental.pallas import tpu_sc as plsc

N = 10000
E = 320000
D = 128
NPAD = 10240
W_EDGES = 128
N_WIN = E // W_EDGES
N_WORKERS = 32
ROWS_PER_TILE = NPAD // 16
PROWS_PER_TILE = NPAD // 16

_mesh = plsc.VectorSubcoreMesh(core_axis_name="c", subcore_axis_name="s")


@functools.partial(
    pl.kernel,
    out_type=jax.ShapeDtypeStruct((NPAD,), jnp.float32),
    mesh=_mesh,
    scratch_types=[
        pltpu.VMEM((W_EDGES,), jnp.int32),
        pltpu.VMEM((W_EDGES,), jnp.float32),
        pltpu.VMEM((PROWS_PER_TILE,), jnp.float32),
        pltpu.VMEM_SHARED((NPAD,), jnp.float32),
    ],
)
def _deg(dst_hbm, deg_hbm, idx_v, ones_v, buf_v, acc_sh):
    c = lax.axis_index("c")
    s = lax.axis_index("s")

    @pl.when(c == 0)
    def _():
        for j in range(W_EDGES // 16):
            ones_v[pl.ds(j * 16, 16)] = jnp.full((16,), 1.0, jnp.float32)

        def zbody(j, _):
            buf_v[pl.ds(j * 16, 16)] = jnp.zeros((16,), jnp.float32)
            return None

        lax.fori_loop(0, PROWS_PER_TILE // 16, zbody, None)
        r0 = s * PROWS_PER_TILE
        pltpu.sync_copy(buf_v, acc_sh.at[pl.ds(r0, PROWS_PER_TILE)])
        plsc.subcore_barrier()

        lo = (N_WIN * s) // 16
        hi = (N_WIN * (s + 1)) // 16

        def body(i, _):
            pltpu.sync_copy(dst_hbm.at[pl.ds(i * W_EDGES, W_EDGES)], idx_v)
            pltpu.sync_copy(ones_v, acc_sh.at[idx_v], add=True)
            return None

        lax.fori_loop(lo, hi, body, None)
        plsc.subcore_barrier()
        pltpu.sync_copy(acc_sh.at[pl.ds(r0, PROWS_PER_TILE)],
                        deg_hbm.at[pl.ds(r0, PROWS_PER_TILE)])


@functools.partial(
    pl.kernel,
    out_type=jax.ShapeDtypeStruct((2, NPAD, D), jnp.float32),
    mesh=_mesh,
    scratch_types=[
        pltpu.VMEM((W_EDGES,), jnp.int32),
        pltpu.VMEM((W_EDGES,), jnp.int32),
        pltpu.VMEM((W_EDGES, D), jnp.float32),
        pltpu.VMEM_SHARED((NPAD, D), jnp.float32),
        pltpu.SemaphoreType.DMA,
    ],
)
def _aggregate(hp_hbm, src_hbm, dst_hbm, p_hbm, src_v, dst_v, rows_v, acc_sh, sem):
    c = lax.axis_index("c")
    s = lax.axis_index("s")
    r0 = s * ROWS_PER_TILE
    pltpu.sync_copy(hp_hbm.at[pl.ds(r0, ROWS_PER_TILE)],
                    acc_sh.at[pl.ds(r0, ROWS_PER_TILE)])
    plsc.subcore_barrier()

    wid = c * 16 + s
    lo = (N_WIN * wid) // N_WORKERS
    hi = (N_WIN * (wid + 1)) // N_WORKERS

    def body(i, _):
        base = i * W_EDGES
        pltpu.sync_copy(src_hbm.at[pl.ds(base, W_EDGES)], src_v)
        pltpu.sync_copy(dst_hbm.at[pl.ds(base, W_EDGES)], dst_v)
        pltpu.async_copy(hp_hbm.at[src_v], rows_v, sem).wait()
        pltpu.sync_copy(rows_v, acc_sh.at[dst_v], add=True)
        return None

    lax.fori_loop(lo, hi, body, None)
    plsc.subcore_barrier()
    pltpu.sync_copy(acc_sh.at[pl.ds(r0, ROWS_PER_TILE)],
                    p_hbm.at[c, pl.ds(r0, ROWS_PER_TILE)])


def _mm1_body(x_ref, deg_ref, w_ref, o_ref):
    dinv = lax.rsqrt(deg_ref[:N] + 1.0)
    o_ref[:N] = jnp.dot(x_ref[...] * dinv, w_ref[...],
                        preferred_element_type=jnp.float32)
    o_ref[N:] = jnp.zeros((NPAD - N, D), jnp.float32)


def _mid_body(p_ref, hp_ref, deg_ref, b_ref, w_ref, o_ref):
    dinv = lax.rsqrt(deg_ref[...] + 1.0)
    agg = p_ref[0] + p_ref[1] - hp_ref[...]
    x2 = jnp.maximum(agg * dinv + b_ref[...], 0.0) * dinv
    o_ref[...] = jnp.dot(x2, w_ref[...], preferred_element_type=jnp.float32)


def _fin_body(p_ref, hp_ref, deg_ref, b_ref, o_ref):
    dinv = lax.rsqrt(deg_ref[:N] + 1.0)
    o_ref[...] = (p_ref[0, :N] + p_ref[1, :N] - hp_ref[:N]) * dinv + b_ref[...]


_out_pad = jax.ShapeDtypeStruct((NPAD, D), jnp.float32)
_mm1 = pl.pallas_call(_mm1_body, out_shape=_out_pad)
_mid = pl.pallas_call(_mid_body, out_shape=_out_pad)
_fin = pl.pallas_call(_fin_body, out_shape=jax.ShapeDtypeStruct((N, D), jnp.float32))


def kernel(x, edge_index, W1, b1, W2, b2):
    src = edge_index[0].astype(jnp.int32)
    dst = edge_index[1].astype(jnp.int32)
    b1r = b1.reshape(1, D)
    b2r = b2.reshape(1, D)

    deg = _deg(dst).reshape(NPAD, 1)
    h1p = _mm1(x, deg, W1)
    p1 = _aggregate(h1p, src, dst)
    h2p = _mid(p1, h1p, deg, b1r, W2)
    p2 = _aggregate(h2p, src, dst)
    return _fin(p2, h2p, deg, b2r)
